# baseline (speedup 1.0000x reference)
import functools

import jax
import jax.numpy as jnp
from jax import lax
from jax.experimental import pallas as pl
from jax.experimental.pallas import tpu as pltpu
from jax.experimental.pallas import tpu_sc as plsc

NC = 2
NS = 16
NW = NC * NS
LANES = 16
CH = 128


def _body(idx_hbm, z_hbm, mean_hbm, logvar_hbm, v_hbm, mu_hbm, lv_hbm,
          idx_v, mu_v, lv_v, z_v, sem_mu, sem_lv, sem_z):
    D = mean_hbm.shape[1]
    n_chunks = idx_v.shape[0]
    wid = lax.axis_index("s") * NC + lax.axis_index("c")
    row0 = wid * n_chunks
    pltpu.sync_copy(idx_hbm.at[pl.ds(row0, n_chunks)], idx_v)
    for c in range(n_chunks):
        base = (row0 + c) * CH
        g_mu = pltpu.async_copy(mean_hbm.at[idx_v.at[c]], mu_v, sem_mu)
        g_lv = pltpu.async_copy(logvar_hbm.at[idx_v.at[c]], lv_v, sem_lv)
        g_z = pltpu.async_copy(z_hbm.at[pl.ds(base, CH)], z_v, sem_z)
        g_mu.wait()
        g_lv.wait()
        g_z.wait()
        o_mu = pltpu.async_copy(mu_v, mu_hbm.at[pl.ds(base, CH)], sem_mu)
        o_lv = pltpu.async_copy(lv_v, lv_hbm.at[pl.ds(base, CH)], sem_lv)

        def row_body(r, carry):
            for k in range(D // LANES):
                sl = pl.ds(k * LANES, LANES)
                z_v[r, sl] = mu_v[r, sl] + jnp.exp(lv_v[r, sl] * 0.5) * z_v[r, sl]
            return carry

        lax.fori_loop(0, CH, row_body, 0)
        o_mu.wait()
        o_lv.wait()
        pltpu.sync_copy(z_v, v_hbm.at[pl.ds(base, CH)])


@jax.jit
def _sc_embed(i2, z2, mean2, logvar2):
    B, D = z2.shape
    n_chunks = B // (NW * CH)
    run = functools.partial(
        pl.kernel,
        out_type=[jax.ShapeDtypeStruct((B, D), jnp.float32)] * 3,
        mesh=plsc.VectorSubcoreMesh(core_axis_name="c", subcore_axis_name="s"),
        scratch_types=[
            pltpu.VMEM((n_chunks, CH), jnp.int32),
            pltpu.VMEM((CH, D), jnp.float32),
            pltpu.VMEM((CH, D), jnp.float32),
            pltpu.VMEM((CH, D), jnp.float32),
            pltpu.SemaphoreType.DMA,
            pltpu.SemaphoreType.DMA,
            pltpu.SemaphoreType.DMA,
        ],
        compiler_params=pltpu.CompilerParams(use_tc_tiling_on_sc=False),
    )(_body)
    return run(i2, z2, mean2, logvar2)


def kernel(i, z, mean, logvar):
    B, W, L = z.shape
    N = mean.shape[0]
    D = W * L
    v2, mu2, lv2 = _sc_embed(
        i.astype(jnp.int32).reshape(B // CH, CH),
        z.reshape(B, D),
        mean.reshape(N, D),
        logvar.reshape(N, D),
    )
    shp = (B, W, L)
    return (v2.reshape(shp), mu2.reshape(shp), lv2.reshape(shp))
